# Initial kernel scaffold; baseline (speedup 1.0000x reference)
import jax
import jax.numpy as jnp
from jax.experimental import pallas as pl


def kernel(feat):
    raise NotImplementedError("write your pallas kernel here")


# speedup vs baseline: 13.2853x; 13.2853x over previous
import functools

import jax
import jax.numpy as jnp
from jax import lax
from jax.experimental import pallas as pl
from jax.experimental.pallas import tpu as pltpu
from jax.experimental.pallas import tpu_sc as plsc

B = 10
N_PER = 10000
K = 100
D = 128

_SC_CORES = 2
_SC_SUBCORES = 16
_NW = _SC_CORES * _SC_SUBCORES
_GATHER_ROWS = 1024
_ROWS_PER_W = _GATHER_ROWS // _NW


def _rowmax_body(x_ref, o_ref):
    o_ref[...] = jnp.max(x_ref[...], axis=1, keepdims=True)


def _rowmax(feat):
    grid = B
    return pl.pallas_call(
        _rowmax_body,
        grid=(grid,),
        in_specs=[pl.BlockSpec((N_PER, D), lambda g: (g, 0))],
        out_specs=pl.BlockSpec((N_PER, 1), lambda g: (g, 0)),
        out_shape=jax.ShapeDtypeStruct((B * N_PER, 1), jnp.float32),
    )(feat)


def _topk_body(keys_ref, idx_ref):
    keys = keys_ref[...]
    lane = lax.broadcasted_iota(jnp.int32, (B, N_PER), 1)
    lane128 = lax.broadcasted_iota(jnp.int32, (B, 128), 1)

    def body(t, carry):
        kc, acc = carry
        m = jnp.max(kc, axis=1, keepdims=True)
        idx = jnp.min(
            jnp.where(kc == m, lane, jnp.int32(1 << 30)), axis=1, keepdims=True
        )
        acc = jnp.where(lane128 == t, idx, acc)
        kc = jnp.where(lane == idx, jnp.float32(-jnp.inf), kc)
        return kc, acc

    _, acc = lax.fori_loop(0, K, body, (keys, lane128))
    row = lax.broadcasted_iota(jnp.int32, (B, 128), 0)
    idx_ref[...] = acc + N_PER * row


def _topk(keys2d):
    return pl.pallas_call(
        _topk_body,
        out_shape=jax.ShapeDtypeStruct((B, 128), jnp.int32),
    )(keys2d)


def _gather_body(feat_hbm, idx_hbm, out_hbm, idx_v, rows_v, sem):
    wid = lax.axis_index("s") * _SC_CORES + lax.axis_index("c")
    base = wid * _ROWS_PER_W
    pltpu.sync_copy(idx_hbm.at[pl.ds(base, _ROWS_PER_W)], idx_v)
    pltpu.async_copy(feat_hbm.at[idx_v], rows_v, sem).wait()
    pltpu.sync_copy(rows_v, out_hbm.at[pl.ds(base, _ROWS_PER_W)])


def _gather_rows(feat, idx_flat):
    mesh = plsc.VectorSubcoreMesh(core_axis_name="c", subcore_axis_name="s")
    kern = functools.partial(
        pl.kernel,
        mesh=mesh,
        out_type=jax.ShapeDtypeStruct((_GATHER_ROWS, D), jnp.float32),
        scratch_types=[
            pltpu.VMEM((_ROWS_PER_W,), jnp.int32),
            pltpu.VMEM((_ROWS_PER_W, D), jnp.float32),
            pltpu.SemaphoreType.DMA,
        ],
    )(_gather_body)
    return kern(feat, idx_flat)


def _rowsort_body(x_ref, o_ref):
    x = x_ref[...]
    i = lax.broadcasted_iota(jnp.int32, x.shape, 1)
    k = 2
    while k <= D:
        j = k // 2
        while j >= 1:
            bitj = (i & j) != 0
            p = jnp.where(bitj, jnp.roll(x, j, axis=1), jnp.roll(x, -j, axis=1))
            want_first = ((i & k) == 0) == ~bitj
            keep = (x <= p) == want_first
            x = jnp.where(keep, x, p)
            j //= 2
        k *= 2
    o_ref[...] = x


def _rowsort(rows):
    return pl.pallas_call(
        _rowsort_body,
        out_shape=jax.ShapeDtypeStruct((_GATHER_ROWS, D), jnp.float32),
    )(rows)


def kernel(feat):
    maxes = _rowmax(feat)
    keys2d = maxes.reshape(B, N_PER)
    idx = _topk(keys2d)
    idx_flat = idx[:, :K].reshape(B * K)
    pad = jnp.arange(_GATHER_ROWS - B * K, dtype=jnp.int32)
    idx_flat = jnp.concatenate([idx_flat, pad])
    rows = _gather_rows(feat, idx_flat)
    srt = _rowsort(rows)
    return srt[: B * K].reshape(B, K * D)
